# Initial kernel scaffold; baseline (speedup 1.0000x reference)
import jax
import jax.numpy as jnp
from jax.experimental import pallas as pl


def kernel(x, edge, W1_0, W1_1, b1, W2_0, W2_1, b2, W3_0, W3_1, b3):
    raise NotImplementedError("write your pallas kernel here")


# speedup vs baseline: 6.0049x; 6.0049x over previous
import functools

import jax
import jax.numpy as jnp
from jax import lax
from jax.experimental import pallas as pl
from jax.experimental.pallas import tpu as pltpu
from jax.experimental.pallas import tpu_sc as plsc

N = 50000
E = 800000
IN = 48
HDIM = 64
FQ = 16
NQ1 = IN // FQ
NQ = HDIM // FQ

NPAD = 50176
EPAD = 819200
EROWS = EPAD // 128
ROWS_PER_TILE = NPAD // 16
TROWS = EROWS // 16
CHUNK = 8

_f32 = jnp.float32
_i32 = jnp.int32


def _sc_mesh():
    return plsc.VectorSubcoreMesh(core_axis_name="c", subcore_axis_name="s",
                                  num_cores=2, num_subcores=16)


DEGW = 16


def _make_degree():
    @functools.partial(
        pl.kernel,
        out_type=(jax.ShapeDtypeStruct((NPAD, DEGW), _f32),
                  jax.ShapeDtypeStruct((NPAD, DEGW), _f32)),
        mesh=_sc_mesh(),
        scratch_types=[
            pltpu.VMEM((CHUNK, 128), _i32),
            pltpu.VMEM((128, DEGW), _f32),
            pltpu.VMEM_SHARED((NPAD, DEGW), _f32),
        ],
        compiler_params=pltpu.CompilerParams(use_tc_tiling_on_sc=False),
    )
    def deg_kernel(src2d, ones_hbm, zeros1_hbm, deg0_hbm, deg1_hbm,
                   sidx, ones_v, acc):
        c = lax.axis_index("c")
        s = lax.axis_index("s")
        r0 = s * ROWS_PER_TILE
        pltpu.sync_copy(zeros1_hbm, acc.at[pl.ds(r0, ROWS_PER_TILE)])
        pltpu.sync_copy(ones_hbm, ones_v)
        plsc.subcore_barrier()

        def run(core, out_hbm):
            base = core * (EROWS // 2) + s * (TROWS // 2)

            def chunk(ch, carry):
                row0 = base + ch * CHUNK
                pltpu.sync_copy(src2d.at[pl.ds(row0, CHUNK)], sidx)
                for j in range(CHUNK):
                    pltpu.sync_copy(ones_v, acc.at[sidx.at[j]], add=True)
                return carry

            lax.fori_loop(0, (TROWS // 2) // CHUNK, chunk, 0)
            plsc.subcore_barrier()
            pltpu.sync_copy(acc.at[pl.ds(r0, ROWS_PER_TILE)],
                            out_hbm.at[pl.ds(r0, ROWS_PER_TILE)])

        @pl.when(c == 0)
        def _():
            run(0, deg0_hbm)

        @pl.when(c == 1)
        def _():
            run(1, deg1_hbm)

    return deg_kernel


def _make_propagate(fq):
    @functools.partial(
        pl.kernel,
        out_type=(jax.ShapeDtypeStruct((NPAD, fq), _f32),
                  jax.ShapeDtypeStruct((NPAD, fq), _f32)),
        mesh=_sc_mesh(),
        scratch_types=[
            pltpu.VMEM((CHUNK, 128), _i32),
            pltpu.VMEM((CHUNK, 128), _i32),
            pltpu.VMEM((CHUNK, 128, fq), _f32),
            pltpu.VMEM_SHARED((NPAD, fq), _f32),
            pltpu.SemaphoreType.DMA,
        ],
        compiler_params=pltpu.CompilerParams(use_tc_tiling_on_sc=False),
    )
    def prop_kernel(src2d, dst2d, ga_hbm, gb_hbm, zeros_hbm,
                    outa_hbm, outb_hbm, sidx, didx, rows, acc, sem):
        c = lax.axis_index("c")
        s = lax.axis_index("s")
        r0 = s * ROWS_PER_TILE
        pltpu.sync_copy(zeros_hbm, acc.at[pl.ds(r0, ROWS_PER_TILE)])
        plsc.subcore_barrier()

        def run(g_hbm, out_hbm):
            base = s * TROWS

            def chunk(ch, carry):
                row0 = base + ch * CHUNK
                pltpu.sync_copy(src2d.at[pl.ds(row0, CHUNK)], sidx)
                pltpu.sync_copy(dst2d.at[pl.ds(row0, CHUNK)], didx)
                for j in range(CHUNK):
                    pltpu.async_copy(g_hbm.at[sidx.at[j]], rows.at[j],
                                     sem).wait()
                    pltpu.sync_copy(rows.at[j], acc.at[didx.at[j]], add=True)
                return carry

            lax.fori_loop(0, TROWS // CHUNK, chunk, 0)
            plsc.subcore_barrier()
            pltpu.sync_copy(acc.at[pl.ds(r0, ROWS_PER_TILE)],
                            out_hbm.at[pl.ds(r0, ROWS_PER_TILE)])

        @pl.when(c == 0)
        def _():
            run(ga_hbm, outa_hbm)

        @pl.when(c == 1)
        def _():
            run(gb_hbm, outb_hbm)

    return prop_kernel


_RB = 512
_GRID = NPAD // _RB


def _full_spec(shape):
    return pl.BlockSpec(shape, lambda i: (0,) * len(shape))


def _row_spec(cols):
    return pl.BlockSpec((_RB, cols), lambda i: (i, 0))


def _make_b1():
    def body(deg0, deg1, x, dis_o, *g_o):
        deg = deg0[...][:, :1] + deg1[...][:, :1]
        dis = jnp.where(deg > 0, lax.rsqrt(jnp.maximum(deg, 1e-12)), 0.0)
        dis_o[...] = dis
        g = dis * x[...]
        for q in range(NQ1):
            g_o[q][...] = g[:, q * FQ:(q + 1) * FQ]

    return pl.pallas_call(
        body,
        grid=(_GRID,),
        in_specs=[_row_spec(DEGW), _row_spec(DEGW), _row_spec(IN)],
        out_specs=(_row_spec(1),) + (_row_spec(FQ),) * NQ1,
        out_shape=(jax.ShapeDtypeStruct((NPAD, 1), _f32),)
        + (jax.ShapeDtypeStruct((NPAD, FQ), _f32),) * NQ1,
    )


def _make_cheb_update(fin, nq, relu, emit_g):
    def body(*refs):
        h, sq, (dis, W0, W1, b) = refs[0], refs[1:1 + nq], refs[1 + nq:5 + nq]
        outs = refs[5 + nq:]
        W1v = W1[...]
        acc = jnp.dot(h[...], W0[...], preferred_element_type=_f32,
                      precision=lax.Precision.HIGHEST)
        for q in range(nq):
            acc -= jnp.dot(dis[...] * sq[q][...], W1v[q * FQ:(q + 1) * FQ],
                           preferred_element_type=_f32,
                           precision=lax.Precision.HIGHEST)
        acc += b[...]
        if relu:
            acc = jnp.maximum(acc, 0.0)
        outs[0][...] = acc
        if emit_g:
            g = dis[...] * acc
            for q in range(NQ):
                outs[1 + q][...] = g[:, q * FQ:(q + 1) * FQ]

    out_specs = [_row_spec(HDIM)]
    out_shape = [jax.ShapeDtypeStruct((NPAD, HDIM), _f32)]
    if emit_g:
        out_specs += [_row_spec(FQ)] * NQ
        out_shape += [jax.ShapeDtypeStruct((NPAD, FQ), _f32)] * NQ

    return pl.pallas_call(
        body,
        grid=(_GRID,),
        in_specs=[_row_spec(fin)] + [_row_spec(FQ)] * nq + [_row_spec(1)]
        + [_full_spec((fin, HDIM)), _full_spec((fin, HDIM)),
           _full_spec((1, HDIM))],
        out_specs=tuple(out_specs),
        out_shape=tuple(out_shape),
    )


_deg_call = _make_degree()
_prop16 = _make_propagate(FQ)
_b1_call = _make_b1()
_c1_call = _make_cheb_update(IN, NQ1, relu=True, emit_g=True)
_c2_call = _make_cheb_update(HDIM, NQ, relu=True, emit_g=True)
_c3_call = _make_cheb_update(HDIM, NQ, relu=False, emit_g=False)


def kernel(x, edge, W1_0, W1_1, b1, W2_0, W2_1, b2, W3_0, W3_1, b3):
    pad_idx = jnp.full((EPAD - E,), N, _i32)
    src2d = jnp.concatenate([edge[0], pad_idx]).reshape(EROWS, 128)
    dst2d = jnp.concatenate([edge[1], pad_idx]).reshape(EROWS, 128)
    x_pad = jnp.pad(x, ((0, NPAD - N), (0, 0)))

    ones = jnp.ones((128, DEGW), _f32)
    zeros16 = jnp.zeros((ROWS_PER_TILE, FQ), _f32)

    deg0, deg1 = _deg_call(src2d, ones, zeros16)
    dis, g1_0, g1_1, g1_2 = _b1_call(deg0, deg1, x_pad)

    def propagate4(gq):
        s0, s1 = _prop16(src2d, dst2d, gq[0], gq[1], zeros16)
        s2, s3 = _prop16(src2d, dst2d, gq[2], gq[3], zeros16)
        return s0, s1, s2, s3

    s1_0, s1_1 = _prop16(src2d, dst2d, g1_0, g1_1, zeros16)
    s1_2, _ = _prop16(src2d, dst2d, g1_2, g1_2, zeros16)
    h1, g2_0, g2_1, g2_2, g2_3 = _c1_call(x_pad, s1_0, s1_1, s1_2, dis,
                                          W1_0, W1_1, b1.reshape(1, HDIM))
    s2q = propagate4((g2_0, g2_1, g2_2, g2_3))
    h2, g3_0, g3_1, g3_2, g3_3 = _c2_call(h1, *s2q, dis, W2_0, W2_1,
                                          b2.reshape(1, HDIM))
    s3q = propagate4((g3_0, g3_1, g3_2, g3_3))
    (h3,) = _c3_call(h2, *s3q, dis, W3_0, W3_1, b3.reshape(1, HDIM))
    return h3[:N]


# speedup vs baseline: 10.1874x; 1.6965x over previous
import functools

import jax
import jax.numpy as jnp
from jax import lax
from jax.experimental import pallas as pl
from jax.experimental.pallas import tpu as pltpu
from jax.experimental.pallas import tpu_sc as plsc

N = 50000
E = 800000
IN = 48
HDIM = 64
FQ = 16
NQ1 = IN // FQ
NQ = HDIM // FQ

NPAD = 50176
EPAD = 819200
EROWS = EPAD // 128
ROWS_PER_TILE = NPAD // 16
TROWS = EROWS // 16
CHUNK = 8

_f32 = jnp.float32
_i32 = jnp.int32


def _sc_mesh():
    return plsc.VectorSubcoreMesh(core_axis_name="c", subcore_axis_name="s",
                                  num_cores=2, num_subcores=16)


DEGW = 16


def _make_degree():
    @functools.partial(
        pl.kernel,
        out_type=(jax.ShapeDtypeStruct((NPAD, DEGW), _f32),
                  jax.ShapeDtypeStruct((NPAD, DEGW), _f32)),
        mesh=_sc_mesh(),
        scratch_types=[
            pltpu.VMEM((CHUNK, 128), _i32),
            pltpu.VMEM((128, DEGW), _f32),
            pltpu.VMEM_SHARED((NPAD, DEGW), _f32),
        ],
        compiler_params=pltpu.CompilerParams(use_tc_tiling_on_sc=False),
    )
    def deg_kernel(src2d, ones_hbm, zeros1_hbm, deg0_hbm, deg1_hbm,
                   sidx, ones_v, acc):
        c = lax.axis_index("c")
        s = lax.axis_index("s")
        r0 = s * ROWS_PER_TILE
        pltpu.sync_copy(zeros1_hbm, acc.at[pl.ds(r0, ROWS_PER_TILE)])
        pltpu.sync_copy(ones_hbm, ones_v)
        plsc.subcore_barrier()

        def run(core, out_hbm):
            base = core * (EROWS // 2) + s * (TROWS // 2)

            def chunk(ch, carry):
                row0 = base + ch * CHUNK
                pltpu.sync_copy(src2d.at[pl.ds(row0, CHUNK)], sidx)
                for j in range(CHUNK):
                    pltpu.sync_copy(ones_v, acc.at[sidx.at[j]], add=True)
                return carry

            lax.fori_loop(0, (TROWS // 2) // CHUNK, chunk, 0)
            plsc.subcore_barrier()
            pltpu.sync_copy(acc.at[pl.ds(r0, ROWS_PER_TILE)],
                            out_hbm.at[pl.ds(r0, ROWS_PER_TILE)])

        @pl.when(c == 0)
        def _():
            run(0, deg0_hbm)

        @pl.when(c == 1)
        def _():
            run(1, deg1_hbm)

    return deg_kernel


def _make_propagate(fq):
    @functools.partial(
        pl.kernel,
        out_type=(jax.ShapeDtypeStruct((NPAD, fq), _f32),
                  jax.ShapeDtypeStruct((NPAD, fq), _f32)),
        mesh=_sc_mesh(),
        scratch_types=[
            pltpu.VMEM((2, CHUNK, 128), _i32),
            pltpu.VMEM((2, CHUNK, 128), _i32),
            pltpu.VMEM((2, CHUNK, 128, fq), _f32),
            pltpu.VMEM_SHARED((NPAD, fq), _f32),
            pltpu.SemaphoreType.DMA,
            pltpu.SemaphoreType.DMA,
            pltpu.SemaphoreType.DMA,
            pltpu.SemaphoreType.DMA,
        ],
        compiler_params=pltpu.CompilerParams(use_tc_tiling_on_sc=False),
    )
    def prop_kernel(src2d, dst2d, ga_hbm, gb_hbm, zeros_hbm,
                    outa_hbm, outb_hbm, sib, dib, rows, acc,
                    sg0, sg1, ss0, ss1):
        c = lax.axis_index("c")
        s = lax.axis_index("s")
        r0 = s * ROWS_PER_TILE
        pltpu.sync_copy(zeros_hbm, acc.at[pl.ds(r0, ROWS_PER_TILE)])
        plsc.subcore_barrier()
        nch = TROWS // CHUNK
        sg = (sg0, sg1)
        ss = (ss0, ss1)

        def run(g_hbm, out_hbm):
            base = s * TROWS

            def load_idx(ch, b):
                row0 = base + ch * CHUNK
                pltpu.sync_copy(src2d.at[pl.ds(row0, CHUNK)], sib.at[b])
                pltpu.sync_copy(dst2d.at[pl.ds(row0, CHUNK)], dib.at[b])

            def gathers(b, fire):
                for j in range(CHUNK):
                    d = pltpu.make_async_copy(g_hbm.at[sib.at[b].at[j]],
                                              rows.at[b].at[j], sg[b])
                    d.start() if fire else d.wait()

            def scatters(b, fire):
                for j in range(CHUNK):
                    if fire:
                        pltpu.async_copy(rows.at[b].at[j],
                                         acc.at[dib.at[b].at[j]], ss[b],
                                         add=True)
                    else:
                        pltpu.make_async_copy(rows.at[b].at[j],
                                              acc.at[dib.at[b].at[j]],
                                              ss[b]).wait()

            def phase(ch, b):
                @pl.when(ch >= 2)
                def _():
                    scatters(b, fire=False)

                load_idx(ch, b)
                gathers(b, fire=True)

                @pl.when(ch >= 1)
                def _():
                    gathers(1 - b, fire=False)
                    scatters(1 - b, fire=True)

            def pair(i, carry):
                phase(2 * i, 0)
                phase(2 * i + 1, 1)
                return carry

            lax.fori_loop(0, nch // 2, pair, 0)
            gathers(1, fire=False)
            scatters(1, fire=True)
            scatters(0, fire=False)
            scatters(1, fire=False)
            plsc.subcore_barrier()
            pltpu.sync_copy(acc.at[pl.ds(r0, ROWS_PER_TILE)],
                            out_hbm.at[pl.ds(r0, ROWS_PER_TILE)])

        @pl.when(c == 0)
        def _():
            run(ga_hbm, outa_hbm)

        @pl.when(c == 1)
        def _():
            run(gb_hbm, outb_hbm)

    return prop_kernel


_RB = 512
_GRID = NPAD // _RB


def _full_spec(shape):
    return pl.BlockSpec(shape, lambda i: (0,) * len(shape))


def _row_spec(cols):
    return pl.BlockSpec((_RB, cols), lambda i: (i, 0))


def _make_b1():
    def body(deg0, deg1, x, dis_o, *g_o):
        deg = deg0[...][:, :1] + deg1[...][:, :1]
        dis = jnp.where(deg > 0, lax.rsqrt(jnp.maximum(deg, 1e-12)), 0.0)
        dis_o[...] = dis
        g = dis * x[...]
        for q in range(NQ1):
            g_o[q][...] = g[:, q * FQ:(q + 1) * FQ]

    return pl.pallas_call(
        body,
        grid=(_GRID,),
        in_specs=[_row_spec(DEGW), _row_spec(DEGW), _row_spec(IN)],
        out_specs=(_row_spec(1),) + (_row_spec(FQ),) * NQ1,
        out_shape=(jax.ShapeDtypeStruct((NPAD, 1), _f32),)
        + (jax.ShapeDtypeStruct((NPAD, FQ), _f32),) * NQ1,
    )


def _make_cheb_update(fin, nq, relu, emit_g):
    def body(*refs):
        h, sq, (dis, W0, W1, b) = refs[0], refs[1:1 + nq], refs[1 + nq:5 + nq]
        outs = refs[5 + nq:]
        W1v = W1[...]
        acc = jnp.dot(h[...], W0[...], preferred_element_type=_f32,
                      precision=lax.Precision.HIGHEST)
        for q in range(nq):
            acc -= jnp.dot(dis[...] * sq[q][...], W1v[q * FQ:(q + 1) * FQ],
                           preferred_element_type=_f32,
                           precision=lax.Precision.HIGHEST)
        acc += b[...]
        if relu:
            acc = jnp.maximum(acc, 0.0)
        outs[0][...] = acc
        if emit_g:
            g = dis[...] * acc
            for q in range(NQ):
                outs[1 + q][...] = g[:, q * FQ:(q + 1) * FQ]

    out_specs = [_row_spec(HDIM)]
    out_shape = [jax.ShapeDtypeStruct((NPAD, HDIM), _f32)]
    if emit_g:
        out_specs += [_row_spec(FQ)] * NQ
        out_shape += [jax.ShapeDtypeStruct((NPAD, FQ), _f32)] * NQ

    return pl.pallas_call(
        body,
        grid=(_GRID,),
        in_specs=[_row_spec(fin)] + [_row_spec(FQ)] * nq + [_row_spec(1)]
        + [_full_spec((fin, HDIM)), _full_spec((fin, HDIM)),
           _full_spec((1, HDIM))],
        out_specs=tuple(out_specs),
        out_shape=tuple(out_shape),
    )


_deg_call = _make_degree()
_prop16 = _make_propagate(FQ)
_b1_call = _make_b1()
_c1_call = _make_cheb_update(IN, NQ1, relu=True, emit_g=True)
_c2_call = _make_cheb_update(HDIM, NQ, relu=True, emit_g=True)
_c3_call = _make_cheb_update(HDIM, NQ, relu=False, emit_g=False)


def kernel(x, edge, W1_0, W1_1, b1, W2_0, W2_1, b2, W3_0, W3_1, b3):
    pad_idx = jnp.full((EPAD - E,), N, _i32)
    src2d = jnp.concatenate([edge[0], pad_idx]).reshape(EROWS, 128)
    dst2d = jnp.concatenate([edge[1], pad_idx]).reshape(EROWS, 128)
    x_pad = jnp.pad(x, ((0, NPAD - N), (0, 0)))

    ones = jnp.ones((128, DEGW), _f32)
    zeros16 = jnp.zeros((ROWS_PER_TILE, FQ), _f32)

    deg0, deg1 = _deg_call(src2d, ones, zeros16)
    dis, g1_0, g1_1, g1_2 = _b1_call(deg0, deg1, x_pad)

    def propagate4(gq):
        s0, s1 = _prop16(src2d, dst2d, gq[0], gq[1], zeros16)
        s2, s3 = _prop16(src2d, dst2d, gq[2], gq[3], zeros16)
        return s0, s1, s2, s3

    s1_0, s1_1 = _prop16(src2d, dst2d, g1_0, g1_1, zeros16)
    s1_2, _ = _prop16(src2d, dst2d, g1_2, g1_2, zeros16)
    h1, g2_0, g2_1, g2_2, g2_3 = _c1_call(x_pad, s1_0, s1_1, s1_2, dis,
                                          W1_0, W1_1, b1.reshape(1, HDIM))
    s2q = propagate4((g2_0, g2_1, g2_2, g2_3))
    h2, g3_0, g3_1, g3_2, g3_3 = _c2_call(h1, *s2q, dis, W2_0, W2_1,
                                          b2.reshape(1, HDIM))
    s3q = propagate4((g3_0, g3_1, g3_2, g3_3))
    (h3,) = _c3_call(h2, *s3q, dis, W3_0, W3_1, b3.reshape(1, HDIM))
    return h3[:N]


# speedup vs baseline: 10.3118x; 1.0122x over previous
import functools

import jax
import jax.numpy as jnp
from jax import lax
from jax.experimental import pallas as pl
from jax.experimental.pallas import tpu as pltpu
from jax.experimental.pallas import tpu_sc as plsc

N = 50000
E = 800000
IN = 48
HDIM = 64
FQ = 16
NQ1 = IN // FQ
NQ = HDIM // FQ

NPAD = 50176
EPAD = 819200
EROWS = EPAD // 128
ROWS_PER_TILE = NPAD // 16
TROWS = EROWS // 16
CHUNK = 10

_f32 = jnp.float32
_i32 = jnp.int32


def _sc_mesh():
    return plsc.VectorSubcoreMesh(core_axis_name="c", subcore_axis_name="s",
                                  num_cores=2, num_subcores=16)


DEGW = 16


def _make_degree():
    @functools.partial(
        pl.kernel,
        out_type=(jax.ShapeDtypeStruct((NPAD, DEGW), _f32),
                  jax.ShapeDtypeStruct((NPAD, DEGW), _f32)),
        mesh=_sc_mesh(),
        scratch_types=[
            pltpu.VMEM((CHUNK, 128), _i32),
            pltpu.VMEM((128, DEGW), _f32),
            pltpu.VMEM_SHARED((NPAD, DEGW), _f32),
        ],
        compiler_params=pltpu.CompilerParams(use_tc_tiling_on_sc=False),
    )
    def deg_kernel(src2d, ones_hbm, zeros1_hbm, deg0_hbm, deg1_hbm,
                   sidx, ones_v, acc):
        c = lax.axis_index("c")
        s = lax.axis_index("s")
        r0 = s * ROWS_PER_TILE
        pltpu.sync_copy(zeros1_hbm, acc.at[pl.ds(r0, ROWS_PER_TILE)])
        pltpu.sync_copy(ones_hbm, ones_v)
        plsc.subcore_barrier()

        def run(core, out_hbm):
            base = core * (EROWS // 2) + s * (TROWS // 2)

            def chunk(ch, carry):
                row0 = base + ch * CHUNK
                pltpu.sync_copy(src2d.at[pl.ds(row0, CHUNK)], sidx)
                for j in range(CHUNK):
                    pltpu.sync_copy(ones_v, acc.at[sidx.at[j]], add=True)
                return carry

            lax.fori_loop(0, (TROWS // 2) // CHUNK, chunk, 0)
            plsc.subcore_barrier()
            pltpu.sync_copy(acc.at[pl.ds(r0, ROWS_PER_TILE)],
                            out_hbm.at[pl.ds(r0, ROWS_PER_TILE)])

        @pl.when(c == 0)
        def _():
            run(0, deg0_hbm)

        @pl.when(c == 1)
        def _():
            run(1, deg1_hbm)

    return deg_kernel


def _make_propagate(fq):
    @functools.partial(
        pl.kernel,
        out_type=(jax.ShapeDtypeStruct((NPAD, fq), _f32),
                  jax.ShapeDtypeStruct((NPAD, fq), _f32)),
        mesh=_sc_mesh(),
        scratch_types=[
            pltpu.VMEM((2, CHUNK, 128), _i32),
            pltpu.VMEM((2, CHUNK, 128), _i32),
            pltpu.VMEM((2, CHUNK, 128, fq), _f32),
            pltpu.VMEM_SHARED((NPAD, fq), _f32),
            pltpu.SemaphoreType.DMA,
            pltpu.SemaphoreType.DMA,
            pltpu.SemaphoreType.DMA,
            pltpu.SemaphoreType.DMA,
        ],
        compiler_params=pltpu.CompilerParams(use_tc_tiling_on_sc=False),
    )
    def prop_kernel(src2d, dst2d, ga_hbm, gb_hbm, zeros_hbm,
                    outa_hbm, outb_hbm, sib, dib, rows, acc,
                    sg0, sg1, ss0, ss1):
        c = lax.axis_index("c")
        s = lax.axis_index("s")
        r0 = s * ROWS_PER_TILE
        pltpu.sync_copy(zeros_hbm, acc.at[pl.ds(r0, ROWS_PER_TILE)])
        plsc.subcore_barrier()
        nch = TROWS // CHUNK
        sg = (sg0, sg1)
        ss = (ss0, ss1)

        def run(g_hbm, out_hbm):
            base = s * TROWS

            def load_idx(ch, b):
                row0 = base + ch * CHUNK
                pltpu.sync_copy(src2d.at[pl.ds(row0, CHUNK)], sib.at[b])
                pltpu.sync_copy(dst2d.at[pl.ds(row0, CHUNK)], dib.at[b])

            def gathers(b, fire):
                for j in range(CHUNK):
                    d = pltpu.make_async_copy(g_hbm.at[sib.at[b].at[j]],
                                              rows.at[b].at[j], sg[b])
                    d.start() if fire else d.wait()

            def scatters(b, fire):
                for j in range(CHUNK):
                    if fire:
                        pltpu.async_copy(rows.at[b].at[j],
                                         acc.at[dib.at[b].at[j]], ss[b],
                                         add=True)
                    else:
                        pltpu.make_async_copy(rows.at[b].at[j],
                                              acc.at[dib.at[b].at[j]],
                                              ss[b]).wait()

            def phase(ch, b):
                @pl.when(ch >= 2)
                def _():
                    scatters(b, fire=False)

                load_idx(ch, b)
                gathers(b, fire=True)

                @pl.when(ch >= 1)
                def _():
                    gathers(1 - b, fire=False)
                    scatters(1 - b, fire=True)

            def pair(i, carry):
                phase(2 * i, 0)
                phase(2 * i + 1, 1)
                return carry

            lax.fori_loop(0, nch // 2, pair, 0)
            gathers(1, fire=False)
            scatters(1, fire=True)
            scatters(0, fire=False)
            scatters(1, fire=False)
            plsc.subcore_barrier()
            pltpu.sync_copy(acc.at[pl.ds(r0, ROWS_PER_TILE)],
                            out_hbm.at[pl.ds(r0, ROWS_PER_TILE)])

        @pl.when(c == 0)
        def _():
            run(ga_hbm, outa_hbm)

        @pl.when(c == 1)
        def _():
            run(gb_hbm, outb_hbm)

    return prop_kernel


_RB = 512
_GRID = NPAD // _RB


def _full_spec(shape):
    return pl.BlockSpec(shape, lambda i: (0,) * len(shape))


def _row_spec(cols):
    return pl.BlockSpec((_RB, cols), lambda i: (i, 0))


def _make_b1():
    def body(deg0, deg1, x, dis_o, *g_o):
        deg = deg0[...][:, :1] + deg1[...][:, :1]
        dis = jnp.where(deg > 0, lax.rsqrt(jnp.maximum(deg, 1e-12)), 0.0)
        dis_o[...] = dis
        g = dis * x[...]
        for q in range(NQ1):
            g_o[q][...] = g[:, q * FQ:(q + 1) * FQ]

    return pl.pallas_call(
        body,
        grid=(_GRID,),
        in_specs=[_row_spec(DEGW), _row_spec(DEGW), _row_spec(IN)],
        out_specs=(_row_spec(1),) + (_row_spec(FQ),) * NQ1,
        out_shape=(jax.ShapeDtypeStruct((NPAD, 1), _f32),)
        + (jax.ShapeDtypeStruct((NPAD, FQ), _f32),) * NQ1,
    )


def _make_cheb_update(fin, nq, relu, emit_g):
    def body(*refs):
        h, sq, (dis, W0, W1, b) = refs[0], refs[1:1 + nq], refs[1 + nq:5 + nq]
        outs = refs[5 + nq:]
        W1v = W1[...]
        acc = jnp.dot(h[...], W0[...], preferred_element_type=_f32,
                      precision=lax.Precision.HIGHEST)
        for q in range(nq):
            acc -= jnp.dot(dis[...] * sq[q][...], W1v[q * FQ:(q + 1) * FQ],
                           preferred_element_type=_f32,
                           precision=lax.Precision.HIGHEST)
        acc += b[...]
        if relu:
            acc = jnp.maximum(acc, 0.0)
        outs[0][...] = acc
        if emit_g:
            g = dis[...] * acc
            for q in range(NQ):
                outs[1 + q][...] = g[:, q * FQ:(q + 1) * FQ]

    out_specs = [_row_spec(HDIM)]
    out_shape = [jax.ShapeDtypeStruct((NPAD, HDIM), _f32)]
    if emit_g:
        out_specs += [_row_spec(FQ)] * NQ
        out_shape += [jax.ShapeDtypeStruct((NPAD, FQ), _f32)] * NQ

    return pl.pallas_call(
        body,
        grid=(_GRID,),
        in_specs=[_row_spec(fin)] + [_row_spec(FQ)] * nq + [_row_spec(1)]
        + [_full_spec((fin, HDIM)), _full_spec((fin, HDIM)),
           _full_spec((1, HDIM))],
        out_specs=tuple(out_specs),
        out_shape=tuple(out_shape),
    )


_deg_call = _make_degree()
_prop16 = _make_propagate(FQ)
_b1_call = _make_b1()
_c1_call = _make_cheb_update(IN, NQ1, relu=True, emit_g=True)
_c2_call = _make_cheb_update(HDIM, NQ, relu=True, emit_g=True)
_c3_call = _make_cheb_update(HDIM, NQ, relu=False, emit_g=False)


def kernel(x, edge, W1_0, W1_1, b1, W2_0, W2_1, b2, W3_0, W3_1, b3):
    pad_idx = jnp.full((EPAD - E,), N, _i32)
    src2d = jnp.concatenate([edge[0], pad_idx]).reshape(EROWS, 128)
    dst2d = jnp.concatenate([edge[1], pad_idx]).reshape(EROWS, 128)
    x_pad = jnp.pad(x, ((0, NPAD - N), (0, 0)))

    ones = jnp.ones((128, DEGW), _f32)
    zeros16 = jnp.zeros((ROWS_PER_TILE, FQ), _f32)

    deg0, deg1 = _deg_call(src2d, ones, zeros16)
    dis, g1_0, g1_1, g1_2 = _b1_call(deg0, deg1, x_pad)

    def propagate4(gq):
        s0, s1 = _prop16(src2d, dst2d, gq[0], gq[1], zeros16)
        s2, s3 = _prop16(src2d, dst2d, gq[2], gq[3], zeros16)
        return s0, s1, s2, s3

    s1_0, s1_1 = _prop16(src2d, dst2d, g1_0, g1_1, zeros16)
    s1_2, _ = _prop16(src2d, dst2d, g1_2, g1_2, zeros16)
    h1, g2_0, g2_1, g2_2, g2_3 = _c1_call(x_pad, s1_0, s1_1, s1_2, dis,
                                          W1_0, W1_1, b1.reshape(1, HDIM))
    s2q = propagate4((g2_0, g2_1, g2_2, g2_3))
    h2, g3_0, g3_1, g3_2, g3_3 = _c2_call(h1, *s2q, dis, W2_0, W2_1,
                                          b2.reshape(1, HDIM))
    s3q = propagate4((g3_0, g3_1, g3_2, g3_3))
    (h3,) = _c3_call(h2, *s3q, dis, W3_0, W3_1, b3.reshape(1, HDIM))
    return h3[:N]


# speedup vs baseline: 15.7324x; 1.5257x over previous
import functools

import jax
import jax.numpy as jnp
from jax import lax
from jax.experimental import pallas as pl
from jax.experimental.pallas import tpu as pltpu
from jax.experimental.pallas import tpu_sc as plsc

N = 50000
E = 800000
IN = 48
HDIM = 64
FH = HDIM // 2

NPAD = 50176
EPAD = 819200
EROWS = EPAD // 128
ROWS_PER_TILE = NPAD // 16
TROWS = EROWS // 16
CHUNK = 10

_f32 = jnp.float32
_bf16 = jnp.bfloat16
_i32 = jnp.int32


def _sc_mesh():
    return plsc.VectorSubcoreMesh(core_axis_name="c", subcore_axis_name="s",
                                  num_cores=2, num_subcores=16)


DEGW = 16


def _make_degree():
    @functools.partial(
        pl.kernel,
        out_type=(jax.ShapeDtypeStruct((NPAD, DEGW), _f32),
                  jax.ShapeDtypeStruct((NPAD, DEGW), _f32)),
        mesh=_sc_mesh(),
        scratch_types=[
            pltpu.VMEM((CHUNK, 128), _i32),
            pltpu.VMEM((128, DEGW), _f32),
            pltpu.VMEM_SHARED((NPAD, DEGW), _f32),
        ],
        compiler_params=pltpu.CompilerParams(use_tc_tiling_on_sc=False),
    )
    def deg_kernel(src2d, ones_hbm, zeros_hbm, deg0_hbm, deg1_hbm,
                   sidx, ones_v, acc):
        c = lax.axis_index("c")
        s = lax.axis_index("s")
        r0 = s * ROWS_PER_TILE
        pltpu.sync_copy(zeros_hbm, acc.at[pl.ds(r0, ROWS_PER_TILE)])
        pltpu.sync_copy(ones_hbm, ones_v)
        plsc.subcore_barrier()

        def run(core, out_hbm):
            base = core * (EROWS // 2) + s * (TROWS // 2)

            def chunk(ch, carry):
                row0 = base + ch * CHUNK
                pltpu.sync_copy(src2d.at[pl.ds(row0, CHUNK)], sidx)
                for j in range(CHUNK):
                    pltpu.sync_copy(ones_v, acc.at[sidx.at[j]], add=True)
                return carry

            lax.fori_loop(0, (TROWS // 2) // CHUNK, chunk, 0)
            plsc.subcore_barrier()
            pltpu.sync_copy(acc.at[pl.ds(r0, ROWS_PER_TILE)],
                            out_hbm.at[pl.ds(r0, ROWS_PER_TILE)])

        @pl.when(c == 0)
        def _():
            run(0, deg0_hbm)

        @pl.when(c == 1)
        def _():
            run(1, deg1_hbm)

    return deg_kernel


def _make_propagate():
    @functools.partial(
        pl.kernel,
        out_type=(jax.ShapeDtypeStruct((NPAD, FH), _bf16),
                  jax.ShapeDtypeStruct((NPAD, FH), _bf16)),
        mesh=_sc_mesh(),
        scratch_types=[
            pltpu.VMEM((2, CHUNK, 128), _i32),
            pltpu.VMEM((2, CHUNK, 128), _i32),
            pltpu.VMEM((2, CHUNK, 128, FH), _bf16),
            pltpu.VMEM_SHARED((NPAD, FH), _bf16),
            pltpu.SemaphoreType.DMA,
            pltpu.SemaphoreType.DMA,
            pltpu.SemaphoreType.DMA,
            pltpu.SemaphoreType.DMA,
        ],
        compiler_params=pltpu.CompilerParams(use_tc_tiling_on_sc=False),
    )
    def prop_kernel(src2d, dst2d, ga_hbm, gb_hbm, zeros_hbm,
                    outa_hbm, outb_hbm, sib, dib, rows, acc,
                    sg0, sg1, ss0, ss1):
        c = lax.axis_index("c")
        s = lax.axis_index("s")
        r0 = s * ROWS_PER_TILE
        pltpu.sync_copy(zeros_hbm, acc.at[pl.ds(r0, ROWS_PER_TILE)])
        plsc.subcore_barrier()
        nch = TROWS // CHUNK
        sg = (sg0, sg1)
        ss = (ss0, ss1)

        def run(g_hbm, out_hbm):
            base = s * TROWS

            def load_idx(ch, b):
                row0 = base + ch * CHUNK
                pltpu.sync_copy(src2d.at[pl.ds(row0, CHUNK)], sib.at[b])
                pltpu.sync_copy(dst2d.at[pl.ds(row0, CHUNK)], dib.at[b])

            def gathers(b, fire):
                for j in range(CHUNK):
                    d = pltpu.make_async_copy(g_hbm.at[sib.at[b].at[j]],
                                              rows.at[b].at[j], sg[b])
                    d.start() if fire else d.wait()

            def scatters(b, fire):
                for j in range(CHUNK):
                    if fire:
                        pltpu.async_copy(rows.at[b].at[j],
                                         acc.at[dib.at[b].at[j]], ss[b],
                                         add=True)
                    else:
                        pltpu.make_async_copy(rows.at[b].at[j],
                                              acc.at[dib.at[b].at[j]],
                                              ss[b]).wait()

            def phase(ch, b):
                @pl.when(ch >= 2)
                def _():
                    scatters(b, fire=False)

                load_idx(ch, b)
                gathers(b, fire=True)

                @pl.when(ch >= 1)
                def _():
                    gathers(1 - b, fire=False)
                    scatters(1 - b, fire=True)

            def pair(i, carry):
                phase(2 * i, 0)
                phase(2 * i + 1, 1)
                return carry

            lax.fori_loop(0, nch // 2, pair, 0)
            gathers(1, fire=False)
            scatters(1, fire=True)
            scatters(0, fire=False)
            scatters(1, fire=False)
            plsc.subcore_barrier()
            pltpu.sync_copy(acc.at[pl.ds(r0, ROWS_PER_TILE)],
                            out_hbm.at[pl.ds(r0, ROWS_PER_TILE)])

        @pl.when(c == 0)
        def _():
            run(ga_hbm, outa_hbm)

        @pl.when(c == 1)
        def _():
            run(gb_hbm, outb_hbm)

    return prop_kernel


_RB = 512
_GRID = NPAD // _RB


def _full_spec(shape):
    return pl.BlockSpec(shape, lambda i: (0,) * len(shape))


def _row_spec(cols):
    return pl.BlockSpec((_RB, cols), lambda i: (i, 0))


def _make_b1():
    def body(deg0, deg1, x, dis_o, ga_o, gb_o):
        deg = deg0[...][:, :1] + deg1[...][:, :1]
        dis = jnp.where(deg > 0, lax.rsqrt(jnp.maximum(deg, 1e-12)), 0.0)
        dis_o[...] = dis
        g = (dis * x[...]).astype(_bf16)
        ga_o[...] = g[:, :FH]
        gb_o[...] = g[:, FH:]

    return pl.pallas_call(
        body,
        grid=(_GRID,),
        in_specs=[_row_spec(DEGW), _row_spec(DEGW), _row_spec(HDIM)],
        out_specs=(_row_spec(1), _row_spec(FH), _row_spec(FH)),
        out_shape=(jax.ShapeDtypeStruct((NPAD, 1), _f32),
                   jax.ShapeDtypeStruct((NPAD, FH), _bf16),
                   jax.ShapeDtypeStruct((NPAD, FH), _bf16)),
    )


def _make_cheb_update(relu, emit_g):
    def body(h, sa, sb, dis, W0, W1, b, *outs):
        W1v = W1[...]
        acc = jnp.dot(h[...], W0[...], preferred_element_type=_f32,
                      precision=lax.Precision.HIGHEST)
        ta = dis[...] * sa[...].astype(_f32)
        tb = dis[...] * sb[...].astype(_f32)
        acc -= jnp.dot(ta, W1v[:FH], preferred_element_type=_f32,
                       precision=lax.Precision.HIGHEST)
        acc -= jnp.dot(tb, W1v[FH:], preferred_element_type=_f32,
                       precision=lax.Precision.HIGHEST)
        acc += b[...]
        if relu:
            acc = jnp.maximum(acc, 0.0)
        outs[0][...] = acc
        if emit_g:
            g = (dis[...] * acc).astype(_bf16)
            outs[1][...] = g[:, :FH]
            outs[2][...] = g[:, FH:]

    out_specs = [_row_spec(HDIM)]
    out_shape = [jax.ShapeDtypeStruct((NPAD, HDIM), _f32)]
    if emit_g:
        out_specs += [_row_spec(FH)] * 2
        out_shape += [jax.ShapeDtypeStruct((NPAD, FH), _bf16)] * 2

    return pl.pallas_call(
        body,
        grid=(_GRID,),
        in_specs=[_row_spec(HDIM), _row_spec(FH), _row_spec(FH), _row_spec(1),
                  _full_spec((HDIM, HDIM)), _full_spec((HDIM, HDIM)),
                  _full_spec((1, HDIM))],
        out_specs=tuple(out_specs),
        out_shape=tuple(out_shape),
    )


_deg_call = _make_degree()
_prop_call = _make_propagate()
_b1_call = _make_b1()
_c1_call = _make_cheb_update(relu=True, emit_g=True)
_c2_call = _make_cheb_update(relu=True, emit_g=True)
_c3_call = _make_cheb_update(relu=False, emit_g=False)


def kernel(x, edge, W1_0, W1_1, b1, W2_0, W2_1, b2, W3_0, W3_1, b3):
    pad_idx = jnp.full((EPAD - E,), N, _i32)
    src2d = jnp.concatenate([edge[0], pad_idx]).reshape(EROWS, 128)
    dst2d = jnp.concatenate([edge[1], pad_idx]).reshape(EROWS, 128)
    x_pad = jnp.pad(x, ((0, NPAD - N), (0, HDIM - IN)))
    W1_0p = jnp.pad(W1_0, ((0, HDIM - IN), (0, 0)))
    W1_1p = jnp.pad(W1_1, ((0, HDIM - IN), (0, 0)))

    ones = jnp.ones((128, DEGW), _f32)
    zeros16 = jnp.zeros((ROWS_PER_TILE, DEGW), _f32)
    zerosb = jnp.zeros((ROWS_PER_TILE, FH), _bf16)

    deg0, deg1 = _deg_call(src2d, ones, zeros16)
    dis, g1a, g1b = _b1_call(deg0, deg1, x_pad)

    s1a, s1b = _prop_call(src2d, dst2d, g1a, g1b, zerosb)
    h1, g2a, g2b = _c1_call(x_pad, s1a, s1b, dis, W1_0p, W1_1p,
                            b1.reshape(1, HDIM))
    s2a, s2b = _prop_call(src2d, dst2d, g2a, g2b, zerosb)
    h2, g3a, g3b = _c2_call(h1, s2a, s2b, dis, W2_0, W2_1,
                            b2.reshape(1, HDIM))
    s3a, s3b = _prop_call(src2d, dst2d, g3a, g3b, zerosb)
    (h3,) = _c3_call(h2, s3a, s3b, dis, W3_0, W3_1, b3.reshape(1, HDIM))
    return h3[:N]


# speedup vs baseline: 21.4703x; 1.3647x over previous
import functools

import jax
import jax.numpy as jnp
from jax import lax
from jax.experimental import pallas as pl
from jax.experimental.pallas import tpu as pltpu
from jax.experimental.pallas import tpu_sc as plsc

N = 50000
E = 800000
IN = 48
HDIM = 64
FH = HDIM // 2

NPAD = 50176
EPAD = 802816
EROWS = EPAD // 128
ROWS_PER_TILE = NPAD // 16
TROWS = EROWS // 16
CHUNK = 14

_f32 = jnp.float32
_bf16 = jnp.bfloat16
_i32 = jnp.int32


def _sc_mesh():
    return plsc.VectorSubcoreMesh(core_axis_name="c", subcore_axis_name="s",
                                  num_cores=2, num_subcores=16)


DEGW = 16


def _make_degree():
    @functools.partial(
        pl.kernel,
        out_type=(jax.ShapeDtypeStruct((NPAD, DEGW), _f32),
                  jax.ShapeDtypeStruct((NPAD, DEGW), _f32)),
        mesh=_sc_mesh(),
        scratch_types=[
            pltpu.VMEM((CHUNK, 128), _i32),
            pltpu.VMEM((128, DEGW), _f32),
            pltpu.VMEM_SHARED((NPAD, DEGW), _f32),
        ],
        compiler_params=pltpu.CompilerParams(use_tc_tiling_on_sc=False),
    )
    def deg_kernel(src2d, ones_hbm, zeros_hbm, deg0_hbm, deg1_hbm,
                   sidx, ones_v, acc):
        c = lax.axis_index("c")
        s = lax.axis_index("s")
        r0 = s * ROWS_PER_TILE
        pltpu.sync_copy(zeros_hbm, acc.at[pl.ds(r0, ROWS_PER_TILE)])
        pltpu.sync_copy(ones_hbm, ones_v)
        plsc.subcore_barrier()

        def run(core, out_hbm):
            base = core * (EROWS // 2) + s * (TROWS // 2)

            def chunk(ch, carry):
                row0 = base + ch * CHUNK
                pltpu.sync_copy(src2d.at[pl.ds(row0, CHUNK)], sidx)
                for j in range(CHUNK):
                    pltpu.sync_copy(ones_v, acc.at[sidx.at[j]], add=True)
                return carry

            lax.fori_loop(0, (TROWS // 2) // CHUNK, chunk, 0)
            plsc.subcore_barrier()
            pltpu.sync_copy(acc.at[pl.ds(r0, ROWS_PER_TILE)],
                            out_hbm.at[pl.ds(r0, ROWS_PER_TILE)])

        @pl.when(c == 0)
        def _():
            run(0, deg0_hbm)

        @pl.when(c == 1)
        def _():
            run(1, deg1_hbm)

    return deg_kernel


def _make_propagate():
    @functools.partial(
        pl.kernel,
        out_type=(jax.ShapeDtypeStruct((NPAD, FH), _bf16),
                  jax.ShapeDtypeStruct((NPAD, FH), _bf16)),
        mesh=_sc_mesh(),
        scratch_types=[
            pltpu.VMEM((2, CHUNK, 128), _i32),
            pltpu.VMEM((2, CHUNK, 128), _i32),
            pltpu.VMEM((2, CHUNK, 128, FH), _bf16),
            pltpu.VMEM_SHARED((NPAD, FH), _bf16),
            pltpu.SemaphoreType.DMA,
            pltpu.SemaphoreType.DMA,
            pltpu.SemaphoreType.DMA,
            pltpu.SemaphoreType.DMA,
        ],
        compiler_params=pltpu.CompilerParams(use_tc_tiling_on_sc=False),
    )
    def prop_kernel(src2d, dst2d, ga_hbm, gb_hbm, zeros_hbm,
                    outa_hbm, outb_hbm, sib, dib, rows, acc,
                    sg0, sg1, ss0, ss1):
        c = lax.axis_index("c")
        s = lax.axis_index("s")
        r0 = s * ROWS_PER_TILE
        pltpu.sync_copy(zeros_hbm, acc.at[pl.ds(r0, ROWS_PER_TILE)])
        plsc.subcore_barrier()
        nch = TROWS // CHUNK
        sg = (sg0, sg1)
        ss = (ss0, ss1)

        def run(g_hbm, out_hbm):
            base = s * TROWS

            def load_idx(ch, b):
                row0 = base + ch * CHUNK
                pltpu.sync_copy(src2d.at[pl.ds(row0, CHUNK)], sib.at[b])
                pltpu.sync_copy(dst2d.at[pl.ds(row0, CHUNK)], dib.at[b])

            def gathers(b, fire):
                for j in range(CHUNK):
                    d = pltpu.make_async_copy(g_hbm.at[sib.at[b].at[j]],
                                              rows.at[b].at[j], sg[b])
                    d.start() if fire else d.wait()

            def scatters(b, fire):
                for j in range(CHUNK):
                    if fire:
                        pltpu.async_copy(rows.at[b].at[j],
                                         acc.at[dib.at[b].at[j]], ss[b],
                                         add=True)
                    else:
                        pltpu.make_async_copy(rows.at[b].at[j],
                                              acc.at[dib.at[b].at[j]],
                                              ss[b]).wait()

            def phase(ch, b):
                @pl.when(ch >= 2)
                def _():
                    scatters(b, fire=False)

                load_idx(ch, b)
                gathers(b, fire=True)

                @pl.when(ch >= 1)
                def _():
                    gathers(1 - b, fire=False)
                    scatters(1 - b, fire=True)

            def pair(i, carry):
                phase(2 * i, 0)
                phase(2 * i + 1, 1)
                return carry

            lax.fori_loop(0, nch // 2, pair, 0)
            gathers(1, fire=False)
            scatters(1, fire=True)
            scatters(0, fire=False)
            scatters(1, fire=False)
            plsc.subcore_barrier()
            pltpu.sync_copy(acc.at[pl.ds(r0, ROWS_PER_TILE)],
                            out_hbm.at[pl.ds(r0, ROWS_PER_TILE)])

        @pl.when(c == 0)
        def _():
            run(ga_hbm, outa_hbm)

        @pl.when(c == 1)
        def _():
            run(gb_hbm, outb_hbm)

    return prop_kernel


_RB = 512
_GRID = NPAD // _RB


def _full_spec(shape):
    return pl.BlockSpec(shape, lambda i: (0,) * len(shape))


def _row_spec(cols):
    return pl.BlockSpec((_RB, cols), lambda i: (i, 0))


def _make_b1():
    def body(deg0, deg1, x, dis_o, ga_o, gb_o):
        deg = deg0[...][:, :1] + deg1[...][:, :1]
        dis = jnp.where(deg > 0, lax.rsqrt(jnp.maximum(deg, 1e-12)), 0.0)
        dis_o[...] = dis
        g = (dis * x[...]).astype(_bf16)
        ga_o[...] = g[:, :FH]
        gb_o[...] = g[:, FH:]

    return pl.pallas_call(
        body,
        grid=(_GRID,),
        in_specs=[_row_spec(DEGW), _row_spec(DEGW), _row_spec(HDIM)],
        out_specs=(_row_spec(1), _row_spec(FH), _row_spec(FH)),
        out_shape=(jax.ShapeDtypeStruct((NPAD, 1), _f32),
                   jax.ShapeDtypeStruct((NPAD, FH), _bf16),
                   jax.ShapeDtypeStruct((NPAD, FH), _bf16)),
    )


def _make_cheb_update(relu, emit_g):
    def body(h, sa, sb, dis, W0, W1, b, *outs):
        W1v = W1[...]
        acc = jnp.dot(h[...], W0[...], preferred_element_type=_f32)
        ta = dis[...] * sa[...].astype(_f32)
        tb = dis[...] * sb[...].astype(_f32)
        acc -= jnp.dot(ta, W1v[:FH], preferred_element_type=_f32)
        acc -= jnp.dot(tb, W1v[FH:], preferred_element_type=_f32)
        acc += b[...]
        if relu:
            acc = jnp.maximum(acc, 0.0)
        outs[0][...] = acc
        if emit_g:
            g = (dis[...] * acc).astype(_bf16)
            outs[1][...] = g[:, :FH]
            outs[2][...] = g[:, FH:]

    out_specs = [_row_spec(HDIM)]
    out_shape = [jax.ShapeDtypeStruct((NPAD, HDIM), _f32)]
    if emit_g:
        out_specs += [_row_spec(FH)] * 2
        out_shape += [jax.ShapeDtypeStruct((NPAD, FH), _bf16)] * 2

    return pl.pallas_call(
        body,
        grid=(_GRID,),
        in_specs=[_row_spec(HDIM), _row_spec(FH), _row_spec(FH), _row_spec(1),
                  _full_spec((HDIM, HDIM)), _full_spec((HDIM, HDIM)),
                  _full_spec((1, HDIM))],
        out_specs=tuple(out_specs),
        out_shape=tuple(out_shape),
    )


_deg_call = _make_degree()
_prop_call = _make_propagate()
_b1_call = _make_b1()
_c1_call = _make_cheb_update(relu=True, emit_g=True)
_c2_call = _make_cheb_update(relu=True, emit_g=True)
_c3_call = _make_cheb_update(relu=False, emit_g=False)


def kernel(x, edge, W1_0, W1_1, b1, W2_0, W2_1, b2, W3_0, W3_1, b3):
    pad_idx = jnp.full((EPAD - E,), N, _i32)
    src2d = jnp.concatenate([edge[0], pad_idx]).reshape(EROWS, 128)
    dst2d = jnp.concatenate([edge[1], pad_idx]).reshape(EROWS, 128)
    x_pad = jnp.pad(x, ((0, NPAD - N), (0, HDIM - IN)))
    W1_0p = jnp.pad(W1_0, ((0, HDIM - IN), (0, 0)))
    W1_1p = jnp.pad(W1_1, ((0, HDIM - IN), (0, 0)))

    ones = jnp.ones((128, DEGW), _f32)
    zeros16 = jnp.zeros((ROWS_PER_TILE, DEGW), _f32)
    zerosb = jnp.zeros((ROWS_PER_TILE, FH), _bf16)

    deg0, deg1 = _deg_call(src2d, ones, zeros16)
    dis, g1a, g1b = _b1_call(deg0, deg1, x_pad)

    s1a, s1b = _prop_call(src2d, dst2d, g1a, g1b, zerosb)
    h1, g2a, g2b = _c1_call(x_pad, s1a, s1b, dis, W1_0p, W1_1p,
                            b1.reshape(1, HDIM))
    s2a, s2b = _prop_call(src2d, dst2d, g2a, g2b, zerosb)
    h2, g3a, g3b = _c2_call(h1, s2a, s2b, dis, W2_0, W2_1,
                            b2.reshape(1, HDIM))
    s3a, s3b = _prop_call(src2d, dst2d, g3a, g3b, zerosb)
    (h3,) = _c3_call(h2, s3a, s3b, dis, W3_0, W3_1, b3.reshape(1, HDIM))
    return h3[:N]


# speedup vs baseline: 25.3372x; 1.1801x over previous
import functools

import jax
import jax.numpy as jnp
from jax import lax
from jax.experimental import pallas as pl
from jax.experimental.pallas import tpu as pltpu
from jax.experimental.pallas import tpu_sc as plsc

N = 50000
E = 800000
IN = 48
HDIM = 64
FH = HDIM // 2

NPAD = 50176
EPAD = 802816
EROWS = EPAD // 128
ROWS_PER_TILE = NPAD // 16
TROWS = EROWS // 16
CHUNK = 14

_f32 = jnp.float32
_bf16 = jnp.bfloat16
_i32 = jnp.int32


def _sc_mesh():
    return plsc.VectorSubcoreMesh(core_axis_name="c", subcore_axis_name="s",
                                  num_cores=2, num_subcores=16)


DEGW = 16


def _make_degree():
    @functools.partial(
        pl.kernel,
        out_type=(jax.ShapeDtypeStruct((NPAD, DEGW), _f32),
                  jax.ShapeDtypeStruct((NPAD, DEGW), _f32)),
        mesh=_sc_mesh(),
        scratch_types=[
            pltpu.VMEM((CHUNK, 128), _i32),
            pltpu.VMEM((128, DEGW), _f32),
            pltpu.VMEM_SHARED((NPAD, DEGW), _f32),
        ],
        compiler_params=pltpu.CompilerParams(use_tc_tiling_on_sc=False),
    )
    def deg_kernel(src2d, ones_hbm, zeros_hbm, deg0_hbm, deg1_hbm,
                   sidx, ones_v, acc):
        c = lax.axis_index("c")
        s = lax.axis_index("s")
        r0 = s * ROWS_PER_TILE
        pltpu.sync_copy(zeros_hbm, acc.at[pl.ds(r0, ROWS_PER_TILE)])
        pltpu.sync_copy(ones_hbm, ones_v)
        plsc.subcore_barrier()

        def run(core, out_hbm):
            base = core * (EROWS // 2) + s * (TROWS // 2)

            def chunk(ch, carry):
                row0 = base + ch * CHUNK
                pltpu.sync_copy(src2d.at[pl.ds(row0, CHUNK)], sidx)
                for j in range(CHUNK):
                    pltpu.sync_copy(ones_v, acc.at[sidx.at[j]], add=True)
                return carry

            lax.fori_loop(0, (TROWS // 2) // CHUNK, chunk, 0)
            plsc.subcore_barrier()
            pltpu.sync_copy(acc.at[pl.ds(r0, ROWS_PER_TILE)],
                            out_hbm.at[pl.ds(r0, ROWS_PER_TILE)])

        @pl.when(c == 0)
        def _():
            run(0, deg0_hbm)

        @pl.when(c == 1)
        def _():
            run(1, deg1_hbm)

    return deg_kernel


def _make_propagate():
    @functools.partial(
        pl.kernel,
        out_type=(jax.ShapeDtypeStruct((NPAD, FH), _bf16),
                  jax.ShapeDtypeStruct((NPAD, FH), _bf16)),
        mesh=_sc_mesh(),
        scratch_types=[
            pltpu.VMEM((2, CHUNK, 128), _i32),
            pltpu.VMEM((2, CHUNK, 128), _i32),
            pltpu.VMEM((2, CHUNK, 128, FH), _bf16),
            pltpu.VMEM_SHARED((NPAD, FH), _bf16),
            pltpu.SemaphoreType.DMA,
            pltpu.SemaphoreType.DMA,
            pltpu.SemaphoreType.DMA,
            pltpu.SemaphoreType.DMA,
        ],
        compiler_params=pltpu.CompilerParams(use_tc_tiling_on_sc=False),
    )
    def prop_kernel(src2d, dst2d, ga_hbm, gb_hbm, zeros_hbm,
                    outa_hbm, outb_hbm, sib, dib, rows, acc,
                    sg0, sg1, ss0, ss1):
        c = lax.axis_index("c")
        s = lax.axis_index("s")
        r0 = s * ROWS_PER_TILE
        pltpu.sync_copy(zeros_hbm, acc.at[pl.ds(r0, ROWS_PER_TILE)])
        plsc.subcore_barrier()
        nch = TROWS // CHUNK
        sg = (sg0, sg1)
        ss = (ss0, ss1)

        def run(g_hbm, out_hbm):
            base = s * TROWS

            def load_idx(ch, b):
                row0 = base + ch * CHUNK
                pltpu.sync_copy(src2d.at[pl.ds(row0, CHUNK)], sib.at[b])
                pltpu.sync_copy(dst2d.at[pl.ds(row0, CHUNK)], dib.at[b])

            def gathers(b, fire):
                for j in range(CHUNK):
                    d = pltpu.make_async_copy(g_hbm.at[sib.at[b].at[j]],
                                              rows.at[b].at[j], sg[b])
                    d.start() if fire else d.wait()

            def scatters(b, fire):
                for j in range(CHUNK):
                    if fire:
                        pltpu.async_copy(rows.at[b].at[j],
                                         acc.at[dib.at[b].at[j]], ss[b],
                                         add=True)
                    else:
                        pltpu.make_async_copy(rows.at[b].at[j],
                                              acc.at[dib.at[b].at[j]],
                                              ss[b]).wait()

            def phase(ch, b):
                @pl.when(ch >= 2)
                def _():
                    scatters(b, fire=False)

                load_idx(ch, b)
                gathers(b, fire=True)

                @pl.when(ch >= 1)
                def _():
                    gathers(1 - b, fire=False)
                    scatters(1 - b, fire=True)

            def pair(i, carry):
                phase(2 * i, 0)
                phase(2 * i + 1, 1)
                return carry

            lax.fori_loop(0, nch // 2, pair, 0)
            gathers(1, fire=False)
            scatters(1, fire=True)
            scatters(0, fire=False)
            scatters(1, fire=False)
            plsc.subcore_barrier()
            pltpu.sync_copy(acc.at[pl.ds(r0, ROWS_PER_TILE)],
                            out_hbm.at[pl.ds(r0, ROWS_PER_TILE)])

        @pl.when(c == 0)
        def _():
            run(ga_hbm, outa_hbm)

        @pl.when(c == 1)
        def _():
            run(gb_hbm, outb_hbm)

    return prop_kernel


_RB = 3136
_GRID = NPAD // _RB
_RB3 = 2000
_GRID3 = N // _RB3


def _full_spec(shape):
    return pl.BlockSpec(shape, lambda i: (0,) * len(shape))


def _row_spec(cols, rb=_RB):
    return pl.BlockSpec((rb, cols), lambda i: (i, 0))


def _dis_of(deg0, deg1):
    deg = deg0[...][:, :1] + deg1[...][:, :1]
    return jnp.where(deg > 0, lax.rsqrt(jnp.maximum(deg, 1e-12)), 0.0)


def _make_b1():
    def body(deg0, deg1, x, ga_o, gb_o):
        g = (_dis_of(deg0, deg1) * x[...]).astype(_bf16)
        ga_o[...] = g[:, :FH]
        gb_o[...] = g[:, FH:]

    return pl.pallas_call(
        body,
        grid=(_GRID,),
        in_specs=[_row_spec(DEGW), _row_spec(DEGW), _row_spec(HDIM)],
        out_specs=(_row_spec(FH), _row_spec(FH)),
        out_shape=(jax.ShapeDtypeStruct((NPAD, FH), _bf16),
                   jax.ShapeDtypeStruct((NPAD, FH), _bf16)),
    )


def _make_cheb_update(relu, emit_g):
    rb, grid, nrows = (_RB, _GRID, NPAD) if emit_g else (_RB3, _GRID3, N)

    def body(h, sa, sb, deg0, deg1, W0, W1, b, *outs):
        dis = _dis_of(deg0, deg1)
        W1v = W1[...]
        acc = jnp.dot(h[...], W0[...], preferred_element_type=_f32)
        ta = dis * sa[...].astype(_f32)
        tb = dis * sb[...].astype(_f32)
        acc -= jnp.dot(ta, W1v[:FH], preferred_element_type=_f32)
        acc -= jnp.dot(tb, W1v[FH:], preferred_element_type=_f32)
        acc += b[...]
        if relu:
            acc = jnp.maximum(acc, 0.0)
        outs[0][...] = acc
        if emit_g:
            g = (dis * acc).astype(_bf16)
            outs[1][...] = g[:, :FH]
            outs[2][...] = g[:, FH:]

    out_specs = [_row_spec(HDIM, rb)]
    out_shape = [jax.ShapeDtypeStruct((nrows, HDIM), _f32)]
    if emit_g:
        out_specs += [_row_spec(FH, rb)] * 2
        out_shape += [jax.ShapeDtypeStruct((NPAD, FH), _bf16)] * 2

    return pl.pallas_call(
        body,
        grid=(grid,),
        in_specs=[_row_spec(HDIM, rb), _row_spec(FH, rb), _row_spec(FH, rb),
                  _row_spec(DEGW, rb), _row_spec(DEGW, rb),
                  _full_spec((HDIM, HDIM)), _full_spec((HDIM, HDIM)),
                  _full_spec((1, HDIM))],
        out_specs=tuple(out_specs),
        out_shape=tuple(out_shape),
    )


_deg_call = _make_degree()
_prop_call = _make_propagate()
_b1_call = _make_b1()
_c1_call = _make_cheb_update(relu=True, emit_g=True)
_c2_call = _make_cheb_update(relu=True, emit_g=True)
_c3_call = _make_cheb_update(relu=False, emit_g=False)


def kernel(x, edge, W1_0, W1_1, b1, W2_0, W2_1, b2, W3_0, W3_1, b3):
    pad_idx = jnp.full((EPAD - E,), N, _i32)
    src2d = jnp.concatenate([edge[0], pad_idx]).reshape(EROWS, 128)
    dst2d = jnp.concatenate([edge[1], pad_idx]).reshape(EROWS, 128)
    x_pad = jnp.pad(x, ((0, NPAD - N), (0, HDIM - IN)))
    W1_0p = jnp.pad(W1_0, ((0, HDIM - IN), (0, 0)))
    W1_1p = jnp.pad(W1_1, ((0, HDIM - IN), (0, 0)))

    ones = jnp.ones((128, DEGW), _f32)
    zeros16 = jnp.zeros((ROWS_PER_TILE, DEGW), _f32)
    zerosb = jnp.zeros((ROWS_PER_TILE, FH), _bf16)

    deg0, deg1 = _deg_call(src2d, ones, zeros16)
    g1a, g1b = _b1_call(deg0, deg1, x_pad)

    s1a, s1b = _prop_call(src2d, dst2d, g1a, g1b, zerosb)
    h1, g2a, g2b = _c1_call(x_pad, s1a, s1b, deg0, deg1, W1_0p, W1_1p,
                            b1.reshape(1, HDIM))
    s2a, s2b = _prop_call(src2d, dst2d, g2a, g2b, zerosb)
    h2, g3a, g3b = _c2_call(h1, s2a, s2b, deg0, deg1, W2_0, W2_1,
                            b2.reshape(1, HDIM))
    s3a, s3b = _prop_call(src2d, dst2d, g3a, g3b, zerosb)
    (h3,) = _c3_call(h2, s3a, s3b, deg0, deg1, W3_0, W3_1,
                     b3.reshape(1, HDIM))
    return h3


# speedup vs baseline: 25.4536x; 1.0046x over previous
import functools

import jax
import jax.numpy as jnp
from jax import lax
from jax.experimental import pallas as pl
from jax.experimental.pallas import tpu as pltpu
from jax.experimental.pallas import tpu_sc as plsc

N = 50000
E = 800000
IN = 48
HDIM = 64
FH = HDIM // 2

NPAD = 50176
EPAD = 802816
EROWS = EPAD // 128
ROWS_PER_TILE = NPAD // 16
TROWS = EROWS // 16
CHUNK = 14

_f32 = jnp.float32
_bf16 = jnp.bfloat16
_i32 = jnp.int32


def _sc_mesh():
    return plsc.VectorSubcoreMesh(core_axis_name="c", subcore_axis_name="s",
                                  num_cores=2, num_subcores=16)


DEGW = 16


def _make_degree():
    @functools.partial(
        pl.kernel,
        out_type=(jax.ShapeDtypeStruct((NPAD, DEGW), _f32),
                  jax.ShapeDtypeStruct((NPAD, DEGW), _f32)),
        mesh=_sc_mesh(),
        scratch_types=[
            pltpu.VMEM((CHUNK, 128), _i32),
            pltpu.VMEM((128, DEGW), _f32),
            pltpu.VMEM_SHARED((NPAD, DEGW), _f32),
        ],
        compiler_params=pltpu.CompilerParams(use_tc_tiling_on_sc=False),
    )
    def deg_kernel(src2d, ones_hbm, zeros_hbm, deg0_hbm, deg1_hbm,
                   sidx, ones_v, acc):
        c = lax.axis_index("c")
        s = lax.axis_index("s")
        r0 = s * ROWS_PER_TILE
        pltpu.sync_copy(zeros_hbm, acc.at[pl.ds(r0, ROWS_PER_TILE)])
        pltpu.sync_copy(ones_hbm, ones_v)
        plsc.subcore_barrier()

        def run(core, out_hbm):
            base = core * (EROWS // 2) + s * (TROWS // 2)

            def chunk(ch, carry):
                row0 = base + ch * CHUNK
                pltpu.sync_copy(src2d.at[pl.ds(row0, CHUNK)], sidx)
                for j in range(CHUNK):
                    pltpu.sync_copy(ones_v, acc.at[sidx.at[j]], add=True)
                return carry

            lax.fori_loop(0, (TROWS // 2) // CHUNK, chunk, 0)
            plsc.subcore_barrier()
            pltpu.sync_copy(acc.at[pl.ds(r0, ROWS_PER_TILE)],
                            out_hbm.at[pl.ds(r0, ROWS_PER_TILE)])

        @pl.when(c == 0)
        def _():
            run(0, deg0_hbm)

        @pl.when(c == 1)
        def _():
            run(1, deg1_hbm)

    return deg_kernel


def _make_propagate():
    @functools.partial(
        pl.kernel,
        out_type=(jax.ShapeDtypeStruct((NPAD, FH), _bf16),
                  jax.ShapeDtypeStruct((NPAD, FH), _bf16)),
        mesh=_sc_mesh(),
        scratch_types=[
            pltpu.VMEM((2, CHUNK, 128), _i32),
            pltpu.VMEM((2, CHUNK, 128), _i32),
            pltpu.VMEM((2, CHUNK, 128, FH), _bf16),
            pltpu.VMEM_SHARED((NPAD, FH), _bf16),
            pltpu.SemaphoreType.DMA,
            pltpu.SemaphoreType.DMA,
            pltpu.SemaphoreType.DMA,
            pltpu.SemaphoreType.DMA,
        ],
        compiler_params=pltpu.CompilerParams(use_tc_tiling_on_sc=False),
    )
    def prop_kernel(src2d, dst2d, ga_hbm, gb_hbm, zeros_hbm,
                    outa_hbm, outb_hbm, sib, dib, rows, acc,
                    sg0, sg1, ss0, ss1):
        c = lax.axis_index("c")
        s = lax.axis_index("s")
        r0 = s * ROWS_PER_TILE
        pltpu.sync_copy(zeros_hbm, acc.at[pl.ds(r0, ROWS_PER_TILE)])
        plsc.subcore_barrier()
        nch = TROWS // CHUNK
        sg = (sg0, sg1)
        ss = (ss0, ss1)

        def run(g_hbm, out_hbm):
            base = s * TROWS

            def load_idx(ch, b):
                row0 = base + ch * CHUNK
                pltpu.sync_copy(src2d.at[pl.ds(row0, CHUNK)], sib.at[b])
                pltpu.sync_copy(dst2d.at[pl.ds(row0, CHUNK)], dib.at[b])

            def gathers(b, fire):
                for j in range(CHUNK):
                    d = pltpu.make_async_copy(g_hbm.at[sib.at[b].at[j]],
                                              rows.at[b].at[j], sg[b])
                    d.start() if fire else d.wait()

            def scatters(b, fire):
                for j in range(CHUNK):
                    if fire:
                        pltpu.async_copy(rows.at[b].at[j],
                                         acc.at[dib.at[b].at[j]], ss[b],
                                         add=True)
                    else:
                        pltpu.make_async_copy(rows.at[b].at[j],
                                              acc.at[dib.at[b].at[j]],
                                              ss[b]).wait()

            def phase(ch, b):
                @pl.when(ch >= 2)
                def _():
                    scatters(b, fire=False)

                load_idx(ch, b)
                gathers(b, fire=True)

                @pl.when(ch >= 1)
                def _():
                    gathers(1 - b, fire=False)
                    scatters(1 - b, fire=True)

            def pair(i, carry):
                phase(2 * i, 0)
                phase(2 * i + 1, 1)
                return carry

            lax.fori_loop(0, nch // 2, pair, 0)
            gathers(1, fire=False)
            scatters(1, fire=True)
            scatters(0, fire=False)
            scatters(1, fire=False)
            plsc.subcore_barrier()
            pltpu.sync_copy(acc.at[pl.ds(r0, ROWS_PER_TILE)],
                            out_hbm.at[pl.ds(r0, ROWS_PER_TILE)])

        @pl.when(c == 0)
        def _():
            run(ga_hbm, outa_hbm)

        @pl.when(c == 1)
        def _():
            run(gb_hbm, outb_hbm)

    return prop_kernel


_RB = 6272
_GRID = NPAD // _RB
_RB3 = 2000
_GRID3 = N // _RB3


def _full_spec(shape):
    return pl.BlockSpec(shape, lambda i: (0,) * len(shape))


def _row_spec(cols, rb=_RB):
    return pl.BlockSpec((rb, cols), lambda i: (i, 0))


def _dis_of(deg0, deg1):
    deg = deg0[...][:, :1] + deg1[...][:, :1]
    return jnp.where(deg > 0, lax.rsqrt(jnp.maximum(deg, 1e-12)), 0.0)


def _make_b1():
    def body(deg0, deg1, x, ga_o, gb_o):
        g = (_dis_of(deg0, deg1) * x[...]).astype(_bf16)
        ga_o[...] = g[:, :FH]
        gb_o[...] = g[:, FH:]

    return pl.pallas_call(
        body,
        grid=(_GRID,),
        in_specs=[_row_spec(DEGW), _row_spec(DEGW), _row_spec(HDIM)],
        out_specs=(_row_spec(FH), _row_spec(FH)),
        out_shape=(jax.ShapeDtypeStruct((NPAD, FH), _bf16),
                   jax.ShapeDtypeStruct((NPAD, FH), _bf16)),
    )


def _make_cheb_update(relu, emit_g):
    rb, grid, nrows = (_RB, _GRID, NPAD) if emit_g else (_RB3, _GRID3, N)

    def body(h, sa, sb, deg0, deg1, W0, W1, b, *outs):
        dis = _dis_of(deg0, deg1)
        W1v = W1[...]
        acc = jnp.dot(h[...], W0[...], preferred_element_type=_f32)
        ta = dis * sa[...].astype(_f32)
        tb = dis * sb[...].astype(_f32)
        acc -= jnp.dot(ta, W1v[:FH], preferred_element_type=_f32)
        acc -= jnp.dot(tb, W1v[FH:], preferred_element_type=_f32)
        acc += b[...]
        if relu:
            acc = jnp.maximum(acc, 0.0)
        outs[0][...] = acc
        if emit_g:
            g = (dis * acc).astype(_bf16)
            outs[1][...] = g[:, :FH]
            outs[2][...] = g[:, FH:]

    out_specs = [_row_spec(HDIM, rb)]
    out_shape = [jax.ShapeDtypeStruct((nrows, HDIM), _f32)]
    if emit_g:
        out_specs += [_row_spec(FH, rb)] * 2
        out_shape += [jax.ShapeDtypeStruct((NPAD, FH), _bf16)] * 2

    return pl.pallas_call(
        body,
        grid=(grid,),
        in_specs=[_row_spec(HDIM, rb), _row_spec(FH, rb), _row_spec(FH, rb),
                  _row_spec(DEGW, rb), _row_spec(DEGW, rb),
                  _full_spec((HDIM, HDIM)), _full_spec((HDIM, HDIM)),
                  _full_spec((1, HDIM))],
        out_specs=tuple(out_specs),
        out_shape=tuple(out_shape),
    )


_deg_call = _make_degree()
_prop_call = _make_propagate()
_b1_call = _make_b1()
_c1_call = _make_cheb_update(relu=True, emit_g=True)
_c2_call = _make_cheb_update(relu=True, emit_g=True)
_c3_call = _make_cheb_update(relu=False, emit_g=False)


def kernel(x, edge, W1_0, W1_1, b1, W2_0, W2_1, b2, W3_0, W3_1, b3):
    pad_idx = jnp.full((EPAD - E,), N, _i32)
    src2d = jnp.concatenate([edge[0], pad_idx]).reshape(EROWS, 128)
    dst2d = jnp.concatenate([edge[1], pad_idx]).reshape(EROWS, 128)
    x_pad = jnp.pad(x, ((0, NPAD - N), (0, HDIM - IN)))
    W1_0p = jnp.pad(W1_0, ((0, HDIM - IN), (0, 0)))
    W1_1p = jnp.pad(W1_1, ((0, HDIM - IN), (0, 0)))

    ones = jnp.ones((128, DEGW), _f32)
    zeros16 = jnp.zeros((ROWS_PER_TILE, DEGW), _f32)
    zerosb = jnp.zeros((ROWS_PER_TILE, FH), _bf16)

    deg0, deg1 = _deg_call(src2d, ones, zeros16)
    g1a, g1b = _b1_call(deg0, deg1, x_pad)

    s1a, s1b = _prop_call(src2d, dst2d, g1a, g1b, zerosb)
    h1, g2a, g2b = _c1_call(x_pad, s1a, s1b, deg0, deg1, W1_0p, W1_1p,
                            b1.reshape(1, HDIM))
    s2a, s2b = _prop_call(src2d, dst2d, g2a, g2b, zerosb)
    h2, g3a, g3b = _c2_call(h1, s2a, s2b, deg0, deg1, W2_0, W2_1,
                            b2.reshape(1, HDIM))
    s3a, s3b = _prop_call(src2d, dst2d, g3a, g3b, zerosb)
    (h3,) = _c3_call(h2, s3a, s3b, deg0, deg1, W3_0, W3_1,
                     b3.reshape(1, HDIM))
    return h3
